# Initial kernel scaffold; baseline (speedup 1.0000x reference)
import jax
import jax.numpy as jnp
from jax.experimental import pallas as pl


def kernel(x, edge_index, W, att_src, att_dst, bias):
    raise NotImplementedError("write your pallas kernel here")


# speedup vs baseline: 5.3819x; 5.3819x over previous
import functools

import jax
import jax.numpy as jnp
from jax import lax
from jax.experimental import pallas as pl
from jax.experimental.pallas import tpu as pltpu
from jax.experimental.pallas import tpu_sc as plsc

N = 10000
IN_CH = 128
OUT_CH = 128

NC = 2
NS = 16
CH = 128
LANES = 16

NP = 2
NQ = NC * NP
Q = 2500
ACC_ROWS = 2560
RPT = ACC_ROWS // NS
DROWS = 32
DRPT = DROWS // NS
APAD = 10016


def _proj_body(x_ref, w_ref, a2_ref, h_ref, a_ref):
    h = jnp.dot(x_ref[...], w_ref[...], preferred_element_type=jnp.float32)
    h_ref[...] = h
    a_ref[...] = jnp.dot(h, a2_ref[...], preferred_element_type=jnp.float32)


def _project(x, W, A2):
    blk = 1000
    grid = N // blk
    return pl.pallas_call(
        _proj_body,
        grid=(grid,),
        in_specs=[
            pl.BlockSpec((blk, IN_CH), lambda i: (i, 0)),
            pl.BlockSpec((IN_CH, OUT_CH), lambda i: (0, 0)),
            pl.BlockSpec((OUT_CH, 128), lambda i: (0, 0)),
        ],
        out_specs=[
            pl.BlockSpec((blk, OUT_CH), lambda i: (i, 0)),
            pl.BlockSpec((blk, 128), lambda i: (i, 0)),
        ],
        out_shape=[
            jax.ShapeDtypeStruct((N, OUT_CH), jnp.float32),
            jax.ShapeDtypeStruct((N, 128), jnp.float32),
        ],
    )(x, W, A2)


def _edge_body(h_hbm, asrc_hbm, adst_hbm, sdi_hbm, zeros_hbm,
               msg_hbm, den_hbm, asrc_v, adst_v, idxs_v, didxloc_v, rows_v,
               denloc_v, idv_v, acc_sh, accd_sh,
               gsem0, gsem1, gsem2, gsem3,
               ssem0, ssem1, ssem2, ssem3,
               isem0, isem1, isem2, isem3):
    c = lax.axis_index("c")
    s = lax.axis_index("s")
    nchunk = sdi_hbm.shape[1] - 4
    r0 = s * RPT
    gsems = (gsem0, gsem1, gsem2, gsem3)
    ssems = (ssem0, ssem1, ssem2, ssem3)
    isems = (isem0, isem1, isem2, isem3)

    pltpu.sync_copy(asrc_hbm, asrc_v)
    pltpu.sync_copy(adst_hbm, adst_v)
    for t in range(DROWS // 16):
        idv_v[pl.ds(t * 16, 16)] = lax.iota(jnp.int32, 16) + t * 16
    zero16 = jnp.zeros((16,), jnp.float32)

    def pass_body(p, _):
        q = NC * p + c
        base = q * Q

        def zinit_body(z, _):
            pltpu.sync_copy(zeros_hbm, acc_sh.at[pl.ds(r0 + z * 32, 32)])
            return 0

        lax.fori_loop(0, RPT // 32, zinit_body, 0)
        pltpu.sync_copy(zeros_hbm.at[pl.ds(0, DRPT)],
                        accd_sh.at[pl.ds(s * DRPT, DRPT)])

        def zero_body(r, _):
            for t in range(8):
                denloc_v[r, pl.ds(t * 16, 16)] = zero16
            return 0

        lax.fori_loop(0, DROWS, zero_body, 0)
        plsc.subcore_barrier()

        for b in range(4):
            pltpu.async_copy(sdi_hbm.at[s, b], idxs_v.at[b], isems[b])
        for b in range(2):
            pltpu.make_async_copy(
                sdi_hbm.at[s, b], idxs_v.at[b], isems[b]).wait()
            pltpu.async_copy(h_hbm.at[idxs_v.at[b, 0]], rows_v.at[b],
                             gsems[b])

        def quad_body(gq, _):
            for b in range(4):
                g = 4 * gq + b
                b2 = (b + 2) % 4
                rw = rows_v.at[b]
                dl = didxloc_v.at[b]
                pltpu.make_async_copy(
                    h_hbm.at[idxs_v.at[b, 0]], rw, gsems[b]).wait()

                def group_body(j, _):
                    si = idxs_v[b, 0, pl.ds(j * LANES, LANES)]
                    di = idxs_v[b, 1, pl.ds(j * LANES, LANES)]
                    a = (plsc.load_gather(asrc_v, [si])
                         + plsc.load_gather(adst_v, [di]))
                    a = jnp.where(a >= 0.0, a, 0.2 * a)
                    e16 = jnp.exp(a)
                    dil = di - base
                    dil = jnp.where((dil >= 0) & (dil < Q), dil, Q)
                    didxloc_v[b, pl.ds(j * LANES, LANES)] = dil
                    plsc.addupdate_scatter(
                        denloc_v,
                        [lax.shift_right_logical(dil, 7),
                         lax.bitwise_and(dil, 127)],
                        e16)
                    for l in range(LANES):
                        je = j * LANES + l
                        eb = jnp.full((16,), e16[l], jnp.float32)
                        for k in range(OUT_CH // 16):
                            rows_v[b, je, pl.ds(k * 16, 16)] = (
                                rows_v[b, je, pl.ds(k * 16, 16)] * eb)
                    return 0

                lax.fori_loop(0, CH // LANES, group_body, 0)

                def wait_sc():
                    pltpu.make_async_copy(
                        rows_v.at[b2], acc_sh.at[didxloc_v.at[b2]],
                        ssems[b2]).wait()

                if b < 2:
                    @pl.when(gq >= 1)
                    def _():
                        wait_sc()
                else:
                    wait_sc()
                pltpu.async_copy(rw, acc_sh.at[dl], ssems[b], add=True)
                pltpu.async_copy(sdi_hbm.at[s, g + 4], idxs_v.at[b],
                                 isems[b])
                pltpu.make_async_copy(
                    sdi_hbm.at[s, g + 2], idxs_v.at[b2], isems[b2]).wait()
                pltpu.async_copy(h_hbm.at[idxs_v.at[b2, 0]], rows_v.at[b2],
                                 gsems[b2])
            return 0

        lax.fori_loop(0, nchunk // 4, quad_body, 0)

        for b in range(2, 4):
            pltpu.make_async_copy(
                rows_v.at[b], acc_sh.at[didxloc_v.at[b]], ssems[b]).wait()
        for b in range(2):
            pltpu.make_async_copy(
                h_hbm.at[idxs_v.at[b, 0]], rows_v.at[b], gsems[b]).wait()
        for b in range(2, 4):
            pltpu.make_async_copy(
                sdi_hbm.at[s, 0], idxs_v.at[b], isems[b]).wait()

        pltpu.sync_copy(denloc_v, accd_sh.at[idv_v], add=True)
        plsc.subcore_barrier()

        pltpu.sync_copy(acc_sh.at[pl.ds(r0, RPT)],
                        msg_hbm.at[q, pl.ds(r0, RPT)])
        pltpu.sync_copy(accd_sh, denloc_v)

        def exp_body(bb, _):
            for t in range(2):
                o = r0 + bb * 32 + t * 16
                dv = denloc_v[lax.shift_right_logical(o, 7),
                              pl.ds(lax.bitwise_and(o, 127), 16)]
                for l in range(LANES):
                    db = jnp.full((16,), dv[l], jnp.float32)
                    for k in range(OUT_CH // 16):
                        rows_v[0, t * 16 + l, pl.ds(k * 16, 16)] = db
            pltpu.sync_copy(rows_v.at[0, pl.ds(0, 32)],
                            den_hbm.at[q, pl.ds(r0 + bb * 32, 32)])
            return 0

        lax.fori_loop(0, RPT // 32, exp_body, 0)
        plsc.subcore_barrier()
        return 0

    lax.fori_loop(0, NP, pass_body, 0)


def _edge_pass(h, asrc, adst, sdi):
    zeros = jnp.zeros((32, OUT_CH), jnp.float32)
    mesh = plsc.VectorSubcoreMesh(core_axis_name="c", subcore_axis_name="s")
    kern = functools.partial(
        pl.kernel,
        mesh=mesh,
        compiler_params=pltpu.CompilerParams(needs_layout_passes=False),
        out_type=[
            jax.ShapeDtypeStruct((NQ, ACC_ROWS, OUT_CH), jnp.float32),
            jax.ShapeDtypeStruct((NQ, ACC_ROWS, OUT_CH), jnp.float32),
        ],
        scratch_types=[
            pltpu.VMEM((APAD,), jnp.float32),
            pltpu.VMEM((APAD,), jnp.float32),
            pltpu.VMEM((4, 2, CH), jnp.int32),
            pltpu.VMEM((4, CH), jnp.int32),
            pltpu.VMEM((4, CH, OUT_CH), jnp.float32),
            pltpu.VMEM((DROWS, 128), jnp.float32),
            pltpu.VMEM((DROWS,), jnp.int32),
            pltpu.VMEM_SHARED((ACC_ROWS, OUT_CH), jnp.float32),
            pltpu.VMEM_SHARED((DROWS, 128), jnp.float32),
        ] + [pltpu.SemaphoreType.DMA] * 12,
    )(_edge_body)
    return kern(h, asrc, adst, sdi, zeros)


def _fin_body(m_ref, d_ref, b_ref, o_ref):
    o_ref[...] = m_ref[...] / (d_ref[...] + 1e-16) + b_ref[...]


def _finalize(msg, den, bias):
    blk = 1024
    rows = msg.shape[0]
    grid = rows // blk
    return pl.pallas_call(
        _fin_body,
        grid=(grid,),
        in_specs=[
            pl.BlockSpec((blk, OUT_CH), lambda i: (i, 0)),
            pl.BlockSpec((blk, OUT_CH), lambda i: (i, 0)),
            pl.BlockSpec((1, OUT_CH), lambda i: (0, 0)),
        ],
        out_specs=pl.BlockSpec((blk, OUT_CH), lambda i: (i, 0)),
        out_shape=jax.ShapeDtypeStruct((rows, OUT_CH), jnp.float32),
    )(msg, den, bias)


def kernel(x, edge_index, W, att_src, att_dst, bias):
    A2 = jnp.zeros((OUT_CH, 128), jnp.float32)
    A2 = A2.at[:, 0].set(att_src[0].astype(jnp.float32))
    A2 = A2.at[:, 1].set(att_dst[0].astype(jnp.float32))

    h, a = _project(x, W, A2)
    asrc = jnp.pad(a[:, 0], (0, APAD - N))
    adst = jnp.pad(a[:, 1], (0, APAD - N))

    ei = edge_index.astype(jnp.int32)
    loop = jnp.arange(N, dtype=jnp.int32)
    e_tot = ei.shape[1] + N
    per = NS * CH
    nchunk = 4 * (-(-e_tot // (4 * per)))
    e_pad = nchunk * per
    src = jnp.concatenate([ei[0], loop, jnp.zeros((e_pad - e_tot,), jnp.int32)])
    dst = jnp.concatenate([ei[1], loop,
                           jnp.full((e_pad - e_tot,), N, jnp.int32)])
    sdi = jnp.stack([src.reshape(NS, nchunk, CH),
                     dst.reshape(NS, nchunk, CH)], axis=2)
    sdi = jnp.pad(sdi, ((0, 0), (0, 4), (0, 0), (0, 0)))

    msg, den = _edge_pass(h, asrc, adst, sdi)
    out = _finalize(msg.reshape(NQ * ACC_ROWS, OUT_CH),
                    den.reshape(NQ * ACC_ROWS, OUT_CH),
                    bias.reshape(1, OUT_CH))
    return jnp.concatenate(
        [out[i * ACC_ROWS:i * ACC_ROWS + Q] for i in range(NQ)])[:N]


# speedup vs baseline: 8.6958x; 1.6157x over previous
import functools

import jax
import jax.numpy as jnp
from jax import lax
from jax.experimental import pallas as pl
from jax.experimental.pallas import tpu as pltpu
from jax.experimental.pallas import tpu_sc as plsc

N = 10000
IN_CH = 128
OUT_CH = 128

NC = 2
NS = 16
CH = 128
LANES = 16

NP = 2
NQ = NC * NP
Q = 2500
ACC_ROWS = 2560
RPT = ACC_ROWS // NS
DROWS = 32
DRPT = DROWS // NS
APAD = 10016


def _proj_body(x_ref, w_ref, a2_ref, h_ref, a_ref):
    h = jnp.dot(x_ref[...], w_ref[...], preferred_element_type=jnp.float32)
    h_ref[...] = h
    a_ref[...] = jnp.dot(h, a2_ref[...], preferred_element_type=jnp.float32)


def _project(x, W, A2):
    blk = 1000
    grid = N // blk
    return pl.pallas_call(
        _proj_body,
        grid=(grid,),
        in_specs=[
            pl.BlockSpec((blk, IN_CH), lambda i: (i, 0)),
            pl.BlockSpec((IN_CH, OUT_CH), lambda i: (0, 0)),
            pl.BlockSpec((OUT_CH, 128), lambda i: (0, 0)),
        ],
        out_specs=[
            pl.BlockSpec((blk, OUT_CH), lambda i: (i, 0)),
            pl.BlockSpec((blk, 128), lambda i: (i, 0)),
        ],
        out_shape=[
            jax.ShapeDtypeStruct((N, OUT_CH), jnp.float32),
            jax.ShapeDtypeStruct((N, 128), jnp.float32),
        ],
    )(x, W, A2)


def _edge_body(h_hbm, asrc_hbm, adst_hbm, srci_hbm, dsti_hbm, zeros_hbm,
               msg_hbm, den_hbm, asrc_v, adst_v, sidx_v, didx_v, didxloc_v,
               rows_v, denloc_v, dloc_v, dexp_v, idv_v, acc_sh, accd_sh,
               gsem0, gsem1):
    c = lax.axis_index("c")
    s = lax.axis_index("s")
    nchunk = srci_hbm.shape[1] - 2
    r0 = s * RPT

    pltpu.sync_copy(asrc_hbm, asrc_v)
    pltpu.sync_copy(adst_hbm, adst_v)
    pltpu.sync_copy(srci_hbm.at[s], sidx_v)
    pltpu.sync_copy(dsti_hbm.at[s], didx_v)
    for t in range(DROWS // 16):
        idv_v[pl.ds(t * 16, 16)] = lax.iota(jnp.int32, 16) + t * 16
    zero16 = jnp.zeros((16,), jnp.float32)

    def pass_body(p, _):
        q = NC * p + c
        base = q * Q

        def zinit_body(z, _):
            pltpu.sync_copy(zeros_hbm, acc_sh.at[pl.ds(r0 + z * 32, 32)])
            return 0

        lax.fori_loop(0, RPT // 32, zinit_body, 0)
        pltpu.sync_copy(zeros_hbm.at[pl.ds(0, DRPT)],
                        accd_sh.at[pl.ds(s * DRPT, DRPT)])

        def zero_body(r, _):
            for t in range(8):
                denloc_v[r, pl.ds(t * 16, 16)] = zero16
            return 0

        lax.fori_loop(0, DROWS, zero_body, 0)
        plsc.subcore_barrier()

        gsems = (gsem0, gsem1)
        pltpu.async_copy(h_hbm.at[sidx_v.at[0]], rows_v.at[0], gsem0)

        def chunk_body(g, _):
            for b in range(2):
                gg = 2 * g + b
                rw = rows_v.at[b]
                dl = didxloc_v.at[b]
                pltpu.async_copy(
                    h_hbm.at[sidx_v.at[gg + 1]], rows_v.at[1 - b],
                    gsems[1 - b])
                pltpu.make_async_copy(
                    h_hbm.at[sidx_v.at[gg]], rw, gsems[b]).wait()

                def group_body(j, _):
                    si = sidx_v[gg, pl.ds(j * LANES, LANES)]
                    di = didx_v[gg, pl.ds(j * LANES, LANES)]
                    a = (plsc.load_gather(asrc_v, [si])
                         + plsc.load_gather(adst_v, [di]))
                    a = jnp.where(a >= 0.0, a, 0.2 * a)
                    e16 = jnp.exp(a)
                    dil = di - base
                    dil = jnp.where((dil >= 0) & (dil < Q), dil, Q)
                    didxloc_v[b, pl.ds(j * LANES, LANES)] = dil
                    plsc.addupdate_scatter(
                        denloc_v,
                        [lax.shift_right_logical(dil, 7),
                         lax.bitwise_and(dil, 127)],
                        e16)
                    for l in range(LANES):
                        je = j * LANES + l
                        eb = jnp.full((16,), e16[l], jnp.float32)
                        for k in range(OUT_CH // 16):
                            rows_v[b, je, pl.ds(k * 16, 16)] = (
                                rows_v[b, je, pl.ds(k * 16, 16)] * eb)
                    return 0

                lax.fori_loop(0, CH // LANES, group_body, 0)
                pltpu.sync_copy(rw, acc_sh.at[dl], add=True)
            return 0

        lax.fori_loop(0, nchunk // 2, chunk_body, 0)
        pltpu.make_async_copy(
            h_hbm.at[sidx_v.at[0]], rows_v.at[0], gsems[0]).wait()

        pltpu.sync_copy(denloc_v, accd_sh.at[idv_v], add=True)
        plsc.subcore_barrier()

        pltpu.sync_copy(acc_sh.at[pl.ds(r0, RPT)],
                        msg_hbm.at[q, pl.ds(r0, RPT)])
        pltpu.sync_copy(accd_sh, dloc_v)

        def exp_body(b, _):
            for t in range(2):
                o = r0 + b * 32 + t * 16
                dv = dloc_v[lax.shift_right_logical(o, 7),
                            pl.ds(lax.bitwise_and(o, 127), 16)]
                for l in range(LANES):
                    db = jnp.full((16,), dv[l], jnp.float32)
                    for k in range(OUT_CH // 16):
                        dexp_v[t * 16 + l, pl.ds(k * 16, 16)] = db
            pltpu.sync_copy(dexp_v, den_hbm.at[q, pl.ds(r0 + b * 32, 32)])
            return 0

        lax.fori_loop(0, RPT // 32, exp_body, 0)
        plsc.subcore_barrier()
        return 0

    lax.fori_loop(0, NP, pass_body, 0)


def _edge_pass(h, asrc, adst, srci, dsti):
    zeros = jnp.zeros((32, OUT_CH), jnp.float32)
    mesh = plsc.VectorSubcoreMesh(core_axis_name="c", subcore_axis_name="s")
    nchunk = srci.shape[1]
    kern = functools.partial(
        pl.kernel,
        mesh=mesh,
        compiler_params=pltpu.CompilerParams(needs_layout_passes=False),
        out_type=[
            jax.ShapeDtypeStruct((NQ, ACC_ROWS, OUT_CH), jnp.float32),
            jax.ShapeDtypeStruct((NQ, ACC_ROWS, OUT_CH), jnp.float32),
        ],
        scratch_types=[
            pltpu.VMEM((APAD,), jnp.float32),
            pltpu.VMEM((APAD,), jnp.float32),
            pltpu.VMEM((nchunk, CH), jnp.int32),
            pltpu.VMEM((nchunk, CH), jnp.int32),
            pltpu.VMEM((2, CH), jnp.int32),
            pltpu.VMEM((2, CH, OUT_CH), jnp.float32),
            pltpu.VMEM((DROWS, 128), jnp.float32),
            pltpu.VMEM((DROWS, 128), jnp.float32),
            pltpu.VMEM((32, OUT_CH), jnp.float32),
            pltpu.VMEM((DROWS,), jnp.int32),
            pltpu.VMEM_SHARED((ACC_ROWS, OUT_CH), jnp.float32),
            pltpu.VMEM_SHARED((DROWS, 128), jnp.float32),
            pltpu.SemaphoreType.DMA,
            pltpu.SemaphoreType.DMA,
        ],
    )(_edge_body)
    return kern(h, asrc, adst, srci, dsti, zeros)


def _fin_body(m_ref, d_ref, b_ref, o_ref):
    o_ref[...] = m_ref[...] / (d_ref[...] + 1e-16) + b_ref[...]


def _finalize(msg, den, bias):
    blk = 1024
    rows = msg.shape[0]
    grid = rows // blk
    return pl.pallas_call(
        _fin_body,
        grid=(grid,),
        in_specs=[
            pl.BlockSpec((blk, OUT_CH), lambda i: (i, 0)),
            pl.BlockSpec((blk, OUT_CH), lambda i: (i, 0)),
            pl.BlockSpec((1, OUT_CH), lambda i: (0, 0)),
        ],
        out_specs=pl.BlockSpec((blk, OUT_CH), lambda i: (i, 0)),
        out_shape=jax.ShapeDtypeStruct((rows, OUT_CH), jnp.float32),
    )(msg, den, bias)


def kernel(x, edge_index, W, att_src, att_dst, bias):
    A2 = jnp.zeros((OUT_CH, 128), jnp.float32)
    A2 = A2.at[:, 0].set(att_src[0].astype(jnp.float32))
    A2 = A2.at[:, 1].set(att_dst[0].astype(jnp.float32))

    h, a = _project(x, W, A2)
    asrc = jnp.pad(a[:, 0], (0, APAD - N))
    adst = jnp.pad(a[:, 1], (0, APAD - N))

    ei = edge_index.astype(jnp.int32)
    loop = jnp.arange(N, dtype=jnp.int32)
    e_tot = ei.shape[1] + N
    per = NS * CH
    nchunk = 2 * (-(-e_tot // (2 * per)))
    e_pad = nchunk * per
    src = jnp.concatenate([ei[0], loop, jnp.zeros((e_pad - e_tot,), jnp.int32)])
    dst = jnp.concatenate([ei[1], loop,
                           jnp.full((e_pad - e_tot,), N, jnp.int32)])
    srci = src.reshape(NS, nchunk, CH)
    dsti = dst.reshape(NS, nchunk, CH)
    srci = jnp.pad(srci, ((0, 0), (0, 2), (0, 0)))
    dsti = jnp.pad(dsti, ((0, 0), (0, 2), (0, 0)), constant_values=N)

    msg, den = _edge_pass(h, asrc, adst, srci, dsti)
    out = _finalize(msg.reshape(NQ * ACC_ROWS, OUT_CH),
                    den.reshape(NQ * ACC_ROWS, OUT_CH),
                    bias.reshape(1, OUT_CH))
    return jnp.concatenate(
        [out[i * ACC_ROWS:i * ACC_ROWS + Q] for i in range(NQ)])[:N]
